# baseline (speedup 1.0000x reference)
import jax
import jax.numpy as jnp
from jax import lax
from jax.experimental import pallas as pl
from jax.experimental.pallas import tpu as pltpu
from jax.experimental.pallas import tpu_sc as plsc

N = 100000
EMB_DIM = 64
D_FEAT = 128
OUT_DIM = EMB_DIM + D_FEAT

R = 160
NB = N // R
NW = 32
MAXJ = -(-NB // NW)
DEPTH = 3


def _sc_body(ids_hbm, feats_hbm, table_hbm, out_hbm, idx_v, emb_v, feats_v,
             *sems):
    wid = lax.axis_index("s") * 2 + lax.axis_index("c")

    def sem(p, k):
        return sems[p * 5 + k]

    def blk(j):
        return wid + j * NW

    def front(j):
        p = j % DEPTH

        @pl.when(blk(j) < NB)
        def _():
            base = blk(j) * R
            pltpu.make_async_copy(
                ids_hbm.at[pl.ds(base, R)], idx_v.at[p], sem(p, 0)).start()
            pltpu.make_async_copy(
                feats_hbm.at[pl.ds(base, R)], feats_v.at[p], sem(p, 2)).start()

    def mid(j):
        p = j % DEPTH

        @pl.when(blk(j) < NB)
        def _():
            pltpu.make_async_copy(
                ids_hbm.at[pl.ds(blk(j) * R, R)], idx_v.at[p],
                sem(p, 0)).wait()
            pltpu.make_async_copy(
                table_hbm.at[idx_v.at[p]], emb_v.at[p], sem(p, 1)).start()

    def back(j):
        p = j % DEPTH

        @pl.when(blk(j) < NB)
        def _():
            base = blk(j) * R
            pltpu.make_async_copy(
                table_hbm.at[idx_v.at[p]], emb_v.at[p], sem(p, 1)).wait()
            pltpu.make_async_copy(
                emb_v.at[p], out_hbm.at[pl.ds(base, R), pl.ds(0, EMB_DIM)],
                sem(p, 3)).start()
            pltpu.make_async_copy(
                feats_hbm.at[pl.ds(base, R)], feats_v.at[p], sem(p, 2)).wait()
            pltpu.make_async_copy(
                feats_v.at[p],
                out_hbm.at[pl.ds(base, R), pl.ds(EMB_DIM, D_FEAT)],
                sem(p, 4)).start()

    def drain(j):
        p = j % DEPTH

        @pl.when(blk(j) < NB)
        def _():
            base = blk(j) * R
            pltpu.make_async_copy(
                emb_v.at[p], out_hbm.at[pl.ds(base, R), pl.ds(0, EMB_DIM)],
                sem(p, 3)).wait()
            pltpu.make_async_copy(
                feats_v.at[p],
                out_hbm.at[pl.ds(base, R), pl.ds(EMB_DIM, D_FEAT)],
                sem(p, 4)).wait()

    for step in range(MAXJ + DEPTH):
        jd = step - DEPTH
        if 0 <= jd < MAXJ:
            drain(jd)
        if step < MAXJ:
            front(step)
        jm = step - 1
        if 0 <= jm < MAXJ:
            mid(jm)
        jb = step - 2
        if 0 <= jb < MAXJ:
            back(jb)


@jax.jit
def _feature_prep(ids, feats, embed_weight):
    mesh = plsc.VectorSubcoreMesh(core_axis_name="c", subcore_axis_name="s")
    return pl.kernel(
        _sc_body,
        mesh=mesh,
        out_type=jax.ShapeDtypeStruct((N, OUT_DIM), jnp.float32),
        scratch_types=[
            pltpu.VMEM((DEPTH, R), jnp.int32),
            pltpu.VMEM((DEPTH, R, EMB_DIM), jnp.float32),
            pltpu.VMEM((DEPTH, R, D_FEAT), jnp.float32),
        ] + [pltpu.SemaphoreType.DMA] * (DEPTH * 5),
        compiler_params=pltpu.CompilerParams(use_tc_tiling_on_sc=False),
    )(ids, feats, embed_weight)


def kernel(ids, feats, embed_weight):
    return _feature_prep(ids.astype(jnp.int32), feats, embed_weight)


# speedup vs baseline: 2.5254x; 2.5254x over previous
import jax
import jax.numpy as jnp
from jax import lax
from jax.experimental import pallas as pl
from jax.experimental.pallas import tpu as pltpu
from jax.experimental.pallas import tpu_sc as plsc

N = 100000
EMB_DIM = 64
D_FEAT = 128
OUT_DIM = EMB_DIM + D_FEAT
P = N // 2

RP = 200
NBP = P // RP
NW = 32
MAXJ = -(-NBP // NW)
DEPTH = 3

BM = 2000
NG = N // BM
HG = NG // 2


def _sc_gather_body(ids_hbm, table_hbm, emb2_hbm, idx_v, lo_v, hi_v, *sems):
    wid = lax.axis_index("s") * 2 + lax.axis_index("c")

    def sem(p, k):
        return sems[p * 5 + k]

    def blk(j):
        return wid + j * NW

    def front(j):
        p = j % DEPTH

        @pl.when(blk(j) < NBP)
        def _():
            base = blk(j) * RP
            pltpu.make_async_copy(
                ids_hbm.at[pl.ds(base, RP)], idx_v.at[p, pl.ds(0, RP)],
                sem(p, 0)).start()
            pltpu.make_async_copy(
                ids_hbm.at[pl.ds(P + base, RP)], idx_v.at[p, pl.ds(RP, RP)],
                sem(p, 0)).start()

    def mid(j):
        p = j % DEPTH

        @pl.when(blk(j) < NBP)
        def _():
            base = blk(j) * RP
            pltpu.make_async_copy(
                ids_hbm.at[pl.ds(base, RP)], idx_v.at[p, pl.ds(0, RP)],
                sem(p, 0)).wait()
            pltpu.make_async_copy(
                ids_hbm.at[pl.ds(P + base, RP)], idx_v.at[p, pl.ds(RP, RP)],
                sem(p, 0)).wait()
            pltpu.make_async_copy(
                table_hbm.at[idx_v.at[p, pl.ds(0, RP)]],
                lo_v.at[p], sem(p, 1)).start()
            pltpu.make_async_copy(
                table_hbm.at[idx_v.at[p, pl.ds(RP, RP)]],
                hi_v.at[p], sem(p, 2)).start()

    def back(j):
        p = j % DEPTH

        @pl.when(blk(j) < NBP)
        def _():
            base = blk(j) * RP
            pltpu.make_async_copy(
                table_hbm.at[idx_v.at[p, pl.ds(0, RP)]],
                lo_v.at[p], sem(p, 1)).wait()
            pltpu.make_async_copy(
                table_hbm.at[idx_v.at[p, pl.ds(RP, RP)]],
                hi_v.at[p], sem(p, 2)).wait()
            pltpu.make_async_copy(
                lo_v.at[p],
                emb2_hbm.at[pl.ds(base, RP), pl.ds(0, EMB_DIM)],
                sem(p, 3)).start()
            pltpu.make_async_copy(
                hi_v.at[p],
                emb2_hbm.at[pl.ds(base, RP), pl.ds(EMB_DIM, EMB_DIM)],
                sem(p, 4)).start()

    def drain(j):
        p = j % DEPTH

        @pl.when(blk(j) < NBP)
        def _():
            base = blk(j) * RP
            pltpu.make_async_copy(
                lo_v.at[p],
                emb2_hbm.at[pl.ds(base, RP), pl.ds(0, EMB_DIM)],
                sem(p, 3)).wait()
            pltpu.make_async_copy(
                hi_v.at[p],
                emb2_hbm.at[pl.ds(base, RP), pl.ds(EMB_DIM, EMB_DIM)],
                sem(p, 4)).wait()

    for step in range(MAXJ + DEPTH):
        jd = step - DEPTH
        if 0 <= jd < MAXJ:
            drain(jd)
        if step < MAXJ:
            front(step)
        jm = step - 1
        if 0 <= jm < MAXJ:
            mid(jm)
        jb = step - 2
        if 0 <= jb < MAXJ:
            back(jb)


def _tc_concat_body(emb2_ref, feats_ref, out_ref):
    i = pl.program_id(0)
    e = emb2_ref[...]
    f = feats_ref[...]

    @pl.when(i < HG)
    def _():
        out_ref[...] = jnp.concatenate([e[:, :EMB_DIM], f], axis=1)

    @pl.when(i >= HG)
    def _():
        out_ref[...] = jnp.concatenate([e[:, EMB_DIM:], f], axis=1)


@jax.jit
def _feature_prep(ids, feats, embed_weight):
    mesh = plsc.VectorSubcoreMesh(core_axis_name="c", subcore_axis_name="s")
    emb2 = pl.kernel(
        _sc_gather_body,
        mesh=mesh,
        out_type=jax.ShapeDtypeStruct((P, 2 * EMB_DIM), jnp.float32),
        scratch_types=[
            pltpu.VMEM((DEPTH, 2 * RP), jnp.int32),
            pltpu.VMEM((DEPTH, RP, EMB_DIM), jnp.float32),
            pltpu.VMEM((DEPTH, RP, EMB_DIM), jnp.float32),
        ] + [pltpu.SemaphoreType.DMA] * (DEPTH * 5),
        compiler_params=pltpu.CompilerParams(use_tc_tiling_on_sc=False),
    )(ids, embed_weight)

    return pl.pallas_call(
        _tc_concat_body,
        grid=(NG,),
        in_specs=[
            pl.BlockSpec((BM, 2 * EMB_DIM),
                         lambda i: (jnp.where(i < HG, i, i - HG), 0)),
            pl.BlockSpec((BM, D_FEAT), lambda i: (i, 0)),
        ],
        out_specs=pl.BlockSpec((BM, OUT_DIM), lambda i: (i, 0)),
        out_shape=jax.ShapeDtypeStruct((N, OUT_DIM), jnp.float32),
    )(emb2, feats)


def kernel(ids, feats, embed_weight):
    return _feature_prep(ids.astype(jnp.int32), feats, embed_weight)


# speedup vs baseline: 2.6007x; 1.0298x over previous
import jax
import jax.numpy as jnp
from jax import lax
from jax.experimental import pallas as pl
from jax.experimental.pallas import tpu as pltpu
from jax.experimental.pallas import tpu_sc as plsc

N = 100000
EMB_DIM = 64
D_FEAT = 128
OUT_DIM = EMB_DIM + D_FEAT

C = 5
CH = N // C
PC = CH // 2

RP = 200
NBP = PC // RP
NW = 32
MAXJ = -(-NBP // NW)
DEPTH = 3

BM = 2000
NG = CH // BM
HG = NG // 2


def _sc_gather_body(ids_hbm, table_hbm, emb2_hbm, idx_v, lo_v, hi_v, *sems):
    wid = lax.axis_index("s") * 2 + lax.axis_index("c")

    def sem(p, k):
        return sems[p * 5 + k]

    def blk(j):
        return wid + j * NW

    def front(j):
        p = j % DEPTH

        @pl.when(blk(j) < NBP)
        def _():
            base = blk(j) * RP
            pltpu.make_async_copy(
                ids_hbm.at[pl.ds(base, RP)], idx_v.at[p, pl.ds(0, RP)],
                sem(p, 0)).start()
            pltpu.make_async_copy(
                ids_hbm.at[pl.ds(PC + base, RP)], idx_v.at[p, pl.ds(RP, RP)],
                sem(p, 0)).start()

    def mid(j):
        p = j % DEPTH

        @pl.when(blk(j) < NBP)
        def _():
            base = blk(j) * RP
            pltpu.make_async_copy(
                ids_hbm.at[pl.ds(base, RP)], idx_v.at[p, pl.ds(0, RP)],
                sem(p, 0)).wait()
            pltpu.make_async_copy(
                ids_hbm.at[pl.ds(PC + base, RP)], idx_v.at[p, pl.ds(RP, RP)],
                sem(p, 0)).wait()
            pltpu.make_async_copy(
                table_hbm.at[idx_v.at[p, pl.ds(0, RP)]],
                lo_v.at[p], sem(p, 1)).start()
            pltpu.make_async_copy(
                table_hbm.at[idx_v.at[p, pl.ds(RP, RP)]],
                hi_v.at[p], sem(p, 2)).start()

    def back(j):
        p = j % DEPTH

        @pl.when(blk(j) < NBP)
        def _():
            base = blk(j) * RP
            pltpu.make_async_copy(
                table_hbm.at[idx_v.at[p, pl.ds(0, RP)]],
                lo_v.at[p], sem(p, 1)).wait()
            pltpu.make_async_copy(
                table_hbm.at[idx_v.at[p, pl.ds(RP, RP)]],
                hi_v.at[p], sem(p, 2)).wait()
            pltpu.make_async_copy(
                lo_v.at[p],
                emb2_hbm.at[pl.ds(base, RP), pl.ds(0, EMB_DIM)],
                sem(p, 3)).start()
            pltpu.make_async_copy(
                hi_v.at[p],
                emb2_hbm.at[pl.ds(base, RP), pl.ds(EMB_DIM, EMB_DIM)],
                sem(p, 4)).start()

    def drain(j):
        p = j % DEPTH

        @pl.when(blk(j) < NBP)
        def _():
            base = blk(j) * RP
            pltpu.make_async_copy(
                lo_v.at[p],
                emb2_hbm.at[pl.ds(base, RP), pl.ds(0, EMB_DIM)],
                sem(p, 3)).wait()
            pltpu.make_async_copy(
                hi_v.at[p],
                emb2_hbm.at[pl.ds(base, RP), pl.ds(EMB_DIM, EMB_DIM)],
                sem(p, 4)).wait()

    for step in range(MAXJ + DEPTH):
        jd = step - DEPTH
        if 0 <= jd < MAXJ:
            drain(jd)
        if step < MAXJ:
            front(step)
        jm = step - 1
        if 0 <= jm < MAXJ:
            mid(jm)
        jb = step - 2
        if 0 <= jb < MAXJ:
            back(jb)


def _sc_gather(ids_chunk, embed_weight):
    mesh = plsc.VectorSubcoreMesh(core_axis_name="c", subcore_axis_name="s")
    return pl.kernel(
        _sc_gather_body,
        mesh=mesh,
        out_type=jax.ShapeDtypeStruct((PC, 2 * EMB_DIM), jnp.float32),
        scratch_types=[
            pltpu.VMEM((DEPTH, 2 * RP), jnp.int32),
            pltpu.VMEM((DEPTH, RP, EMB_DIM), jnp.float32),
            pltpu.VMEM((DEPTH, RP, EMB_DIM), jnp.float32),
        ] + [pltpu.SemaphoreType.DMA] * (DEPTH * 5),
        compiler_params=pltpu.CompilerParams(use_tc_tiling_on_sc=False),
    )(ids_chunk, embed_weight)


def _tc_concat_body(emb2_ref, feats_ref, _, out_ref):
    i = pl.program_id(0)
    e = emb2_ref[...]
    f = feats_ref[...]

    @pl.when(i < HG)
    def _():
        out_ref[...] = jnp.concatenate([e[:, :EMB_DIM], f], axis=1)

    @pl.when(i >= HG)
    def _():
        out_ref[...] = jnp.concatenate([e[:, EMB_DIM:], f], axis=1)


def _tc_concat(c, emb2, feats, out_prev):
    row0 = c * NG
    in_specs = [
        pl.BlockSpec((BM, 2 * EMB_DIM),
                     lambda i: (jnp.where(i < HG, i, i - HG), 0)),
        pl.BlockSpec((BM, D_FEAT), lambda i: (row0 + i, 0)),
    ]
    args = [emb2, feats]
    alias = {}
    if out_prev is not None:
        in_specs.append(pl.BlockSpec(memory_space=pl.ANY))
        args.append(out_prev)
        alias = {2: 0}
    return pl.pallas_call(
        _tc_concat_body if out_prev is not None else
        (lambda e, f, o: _tc_concat_body(e, f, None, o)),
        grid=(NG,),
        in_specs=in_specs,
        out_specs=pl.BlockSpec((BM, OUT_DIM), lambda i: (row0 + i, 0)),
        out_shape=jax.ShapeDtypeStruct((N, OUT_DIM), jnp.float32),
        input_output_aliases=alias,
    )(*args)


@jax.jit
def _feature_prep(ids, feats, embed_weight):
    emb2 = [_sc_gather(ids[c * CH:(c + 1) * CH], embed_weight)
            for c in range(C)]
    out = None
    for c in range(C):
        out = _tc_concat(c, emb2[c], feats, out)
    return out


def kernel(ids, feats, embed_weight):
    return _feature_prep(ids.astype(jnp.int32), feats, embed_weight)


# speedup vs baseline: 2.6825x; 1.0315x over previous
import jax
import jax.numpy as jnp
from jax import lax
from jax.experimental import pallas as pl
from jax.experimental.pallas import tpu as pltpu
from jax.experimental.pallas import tpu_sc as plsc

N = 100000
EMB_DIM = 64
D_FEAT = 128
OUT_DIM = EMB_DIM + D_FEAT

C = 5
CH = N // C
PC = CH // 2

RP = 200
NBP = PC // RP
NW = 32
MAXJ = -(-NBP // NW)
DEPTH = 3

BM = 5000
NG = CH // BM
HG = NG // 2


def _sc_gather_body(ids_hbm, table_hbm, emb2_hbm, idx_v, lo_v, hi_v, *sems):
    wid = lax.axis_index("s") * 2 + lax.axis_index("c")

    def sem(p, k):
        return sems[p * 5 + k]

    def blk(j):
        return wid + j * NW

    def front(j):
        p = j % DEPTH

        @pl.when(blk(j) < NBP)
        def _():
            base = blk(j) * RP
            pltpu.make_async_copy(
                ids_hbm.at[pl.ds(base, RP)], idx_v.at[p, pl.ds(0, RP)],
                sem(p, 0)).start()
            pltpu.make_async_copy(
                ids_hbm.at[pl.ds(PC + base, RP)], idx_v.at[p, pl.ds(RP, RP)],
                sem(p, 0)).start()

    def mid(j):
        p = j % DEPTH

        @pl.when(blk(j) < NBP)
        def _():
            base = blk(j) * RP
            pltpu.make_async_copy(
                ids_hbm.at[pl.ds(base, RP)], idx_v.at[p, pl.ds(0, RP)],
                sem(p, 0)).wait()
            pltpu.make_async_copy(
                ids_hbm.at[pl.ds(PC + base, RP)], idx_v.at[p, pl.ds(RP, RP)],
                sem(p, 0)).wait()
            pltpu.make_async_copy(
                table_hbm.at[idx_v.at[p, pl.ds(0, RP)]],
                lo_v.at[p], sem(p, 1)).start()
            pltpu.make_async_copy(
                table_hbm.at[idx_v.at[p, pl.ds(RP, RP)]],
                hi_v.at[p], sem(p, 2)).start()

    def back(j):
        p = j % DEPTH

        @pl.when(blk(j) < NBP)
        def _():
            base = blk(j) * RP
            pltpu.make_async_copy(
                table_hbm.at[idx_v.at[p, pl.ds(0, RP)]],
                lo_v.at[p], sem(p, 1)).wait()
            pltpu.make_async_copy(
                table_hbm.at[idx_v.at[p, pl.ds(RP, RP)]],
                hi_v.at[p], sem(p, 2)).wait()
            pltpu.make_async_copy(
                lo_v.at[p],
                emb2_hbm.at[pl.ds(base, RP), pl.ds(0, EMB_DIM)],
                sem(p, 3)).start()
            pltpu.make_async_copy(
                hi_v.at[p],
                emb2_hbm.at[pl.ds(base, RP), pl.ds(EMB_DIM, EMB_DIM)],
                sem(p, 4)).start()

    def drain(j):
        p = j % DEPTH

        @pl.when(blk(j) < NBP)
        def _():
            base = blk(j) * RP
            pltpu.make_async_copy(
                lo_v.at[p],
                emb2_hbm.at[pl.ds(base, RP), pl.ds(0, EMB_DIM)],
                sem(p, 3)).wait()
            pltpu.make_async_copy(
                hi_v.at[p],
                emb2_hbm.at[pl.ds(base, RP), pl.ds(EMB_DIM, EMB_DIM)],
                sem(p, 4)).wait()

    for step in range(MAXJ + DEPTH):
        jd = step - DEPTH
        if 0 <= jd < MAXJ:
            drain(jd)
        if step < MAXJ:
            front(step)
        jm = step - 1
        if 0 <= jm < MAXJ:
            mid(jm)
        jb = step - 2
        if 0 <= jb < MAXJ:
            back(jb)


def _sc_gather(ids_chunk, embed_weight):
    mesh = plsc.VectorSubcoreMesh(core_axis_name="c", subcore_axis_name="s")
    return pl.kernel(
        _sc_gather_body,
        mesh=mesh,
        out_type=jax.ShapeDtypeStruct((PC, 2 * EMB_DIM), jnp.float32),
        scratch_types=[
            pltpu.VMEM((DEPTH, 2 * RP), jnp.int32),
            pltpu.VMEM((DEPTH, RP, EMB_DIM), jnp.float32),
            pltpu.VMEM((DEPTH, RP, EMB_DIM), jnp.float32),
        ] + [pltpu.SemaphoreType.DMA] * (DEPTH * 5),
        compiler_params=pltpu.CompilerParams(use_tc_tiling_on_sc=False),
    )(ids_chunk, embed_weight)


def _tc_concat_body(emb2_ref, feats_ref, _, out_ref):
    i = pl.program_id(0)
    e = emb2_ref[...]
    f = feats_ref[...]

    @pl.when(i < HG)
    def _():
        out_ref[...] = jnp.concatenate([e[:, :EMB_DIM], f], axis=1)

    @pl.when(i >= HG)
    def _():
        out_ref[...] = jnp.concatenate([e[:, EMB_DIM:], f], axis=1)


def _tc_concat(c, emb2, feats, out_prev):
    row0 = c * NG
    in_specs = [
        pl.BlockSpec((BM, 2 * EMB_DIM),
                     lambda i: (jnp.where(i < HG, i, i - HG), 0)),
        pl.BlockSpec((BM, D_FEAT), lambda i: (row0 + i, 0)),
    ]
    args = [emb2, feats]
    alias = {}
    if out_prev is not None:
        in_specs.append(pl.BlockSpec(memory_space=pl.ANY))
        args.append(out_prev)
        alias = {2: 0}
    return pl.pallas_call(
        _tc_concat_body if out_prev is not None else
        (lambda e, f, o: _tc_concat_body(e, f, None, o)),
        grid=(NG,),
        in_specs=in_specs,
        out_specs=pl.BlockSpec((BM, OUT_DIM), lambda i: (row0 + i, 0)),
        out_shape=jax.ShapeDtypeStruct((N, OUT_DIM), jnp.float32),
        input_output_aliases=alias,
    )(*args)


@jax.jit
def _feature_prep(ids, feats, embed_weight):
    emb2 = [_sc_gather(ids[c * CH:(c + 1) * CH], embed_weight)
            for c in range(C)]
    out = None
    for c in range(C):
        out = _tc_concat(c, emb2[c], feats, out)
    return out


def kernel(ids, feats, embed_weight):
    return _feature_prep(ids.astype(jnp.int32), feats, embed_weight)


# speedup vs baseline: 4.2598x; 1.5880x over previous
import jax
import jax.numpy as jnp
from jax import lax
from jax.experimental import pallas as pl
from jax.experimental.pallas import tpu as pltpu
from jax.experimental.pallas import tpu_sc as plsc

N = 100000
EMB_DIM = 64
D_FEAT = 128
OUT_DIM = EMB_DIM + D_FEAT
BM = 2048
HB = BM // 2
NG = -(-N // BM)
NP = NG * BM
PP = NP // 2

RP = 256
NBP = PP // RP
NW = 32
MAXJ = -(-NBP // NW)
DEPTH = 3


def _sc_gather_body(ids_e_hbm, ids_o_hbm, table_hbm, emb2_hbm,
                    idx_v, lo_v, hi_v, *sems):
    wid = lax.axis_index("s") * 2 + lax.axis_index("c")

    def sem(p, k):
        return sems[p * 5 + k]

    def blk(j):
        return wid + j * NW

    def front(j):
        p = j % DEPTH

        @pl.when(blk(j) < NBP)
        def _():
            base = blk(j) * RP
            pltpu.make_async_copy(
                ids_e_hbm.at[pl.ds(base, RP)], idx_v.at[p, pl.ds(0, RP)],
                sem(p, 0)).start()
            pltpu.make_async_copy(
                ids_o_hbm.at[pl.ds(base, RP)], idx_v.at[p, pl.ds(RP, RP)],
                sem(p, 0)).start()

    def mid(j):
        p = j % DEPTH

        @pl.when(blk(j) < NBP)
        def _():
            base = blk(j) * RP
            pltpu.make_async_copy(
                ids_e_hbm.at[pl.ds(base, RP)], idx_v.at[p, pl.ds(0, RP)],
                sem(p, 0)).wait()
            pltpu.make_async_copy(
                ids_o_hbm.at[pl.ds(base, RP)], idx_v.at[p, pl.ds(RP, RP)],
                sem(p, 0)).wait()
            pltpu.make_async_copy(
                table_hbm.at[idx_v.at[p, pl.ds(0, RP)]],
                lo_v.at[p], sem(p, 1)).start()
            pltpu.make_async_copy(
                table_hbm.at[idx_v.at[p, pl.ds(RP, RP)]],
                hi_v.at[p], sem(p, 2)).start()

    def back(j):
        p = j % DEPTH

        @pl.when(blk(j) < NBP)
        def _():
            base = blk(j) * RP
            pltpu.make_async_copy(
                table_hbm.at[idx_v.at[p, pl.ds(0, RP)]],
                lo_v.at[p], sem(p, 1)).wait()
            pltpu.make_async_copy(
                table_hbm.at[idx_v.at[p, pl.ds(RP, RP)]],
                hi_v.at[p], sem(p, 2)).wait()
            pltpu.make_async_copy(
                lo_v.at[p],
                emb2_hbm.at[pl.ds(base, RP), pl.ds(0, EMB_DIM)],
                sem(p, 3)).start()
            pltpu.make_async_copy(
                hi_v.at[p],
                emb2_hbm.at[pl.ds(base, RP), pl.ds(EMB_DIM, EMB_DIM)],
                sem(p, 4)).start()

    def drain(j):
        p = j % DEPTH

        @pl.when(blk(j) < NBP)
        def _():
            base = blk(j) * RP
            pltpu.make_async_copy(
                lo_v.at[p],
                emb2_hbm.at[pl.ds(base, RP), pl.ds(0, EMB_DIM)],
                sem(p, 3)).wait()
            pltpu.make_async_copy(
                hi_v.at[p],
                emb2_hbm.at[pl.ds(base, RP), pl.ds(EMB_DIM, EMB_DIM)],
                sem(p, 4)).wait()

    for step in range(MAXJ + DEPTH):
        jd = step - DEPTH
        if 0 <= jd < MAXJ:
            drain(jd)
        if step < MAXJ:
            front(step)
        jm = step - 1
        if 0 <= jm < MAXJ:
            mid(jm)
        jb = step - 2
        if 0 <= jb < MAXJ:
            back(jb)


def _sc_gather(ids_e, ids_o, embed_weight):
    mesh = plsc.VectorSubcoreMesh(core_axis_name="c", subcore_axis_name="s")
    return pl.kernel(
        _sc_gather_body,
        mesh=mesh,
        out_type=jax.ShapeDtypeStruct((PP, 2 * EMB_DIM), jnp.float32),
        scratch_types=[
            pltpu.VMEM((DEPTH, 2 * RP), jnp.int32),
            pltpu.VMEM((DEPTH, RP, EMB_DIM), jnp.float32),
            pltpu.VMEM((DEPTH, RP, EMB_DIM), jnp.float32),
        ] + [pltpu.SemaphoreType.DMA] * (DEPTH * 5),
        compiler_params=pltpu.CompilerParams(use_tc_tiling_on_sc=False),
    )(ids_e, ids_o, embed_weight)


def _tc_concat_body(emb2_ref, feats_ref, out_ref):
    e = emb2_ref[...]
    f = feats_ref[...]
    et = e.T
    emb_part = jnp.concatenate(
        [et[:EMB_DIM, :], et[EMB_DIM:, :]], axis=1)
    out_ref[...] = jnp.concatenate(
        [emb_part, f.T], axis=0)


@jax.jit
def _feature_prep(ids, feats, embed_weight):
    idsb = jnp.pad(ids, (0, NP - N)).reshape(NG, BM)
    lo = idsb[:, :HB].reshape(-1)
    hi = idsb[:, HB:].reshape(-1)
    emb2 = _sc_gather(lo, hi, embed_weight)
    out_t = pl.pallas_call(
        _tc_concat_body,
        grid=(NG,),
        in_specs=[
            pl.BlockSpec((HB, 2 * EMB_DIM), lambda i: (i, 0)),
            pl.BlockSpec((BM, D_FEAT), lambda i: (i, 0)),
        ],
        out_specs=pl.BlockSpec((OUT_DIM, BM), lambda i: (0, i)),
        out_shape=jax.ShapeDtypeStruct((OUT_DIM, N), jnp.float32),
    )(emb2, feats)
    return out_t.T


def kernel(ids, feats, embed_weight):
    return _feature_prep(ids.astype(jnp.int32), feats, embed_weight)


# speedup vs baseline: 4.2696x; 1.0023x over previous
import jax
import jax.numpy as jnp
from jax import lax
from jax.experimental import pallas as pl
from jax.experimental.pallas import tpu as pltpu
from jax.experimental.pallas import tpu_sc as plsc

N = 100000
EMB_DIM = 64
D_FEAT = 128
OUT_DIM = EMB_DIM + D_FEAT
BM = 2048
HB = BM // 2
NG = -(-N // BM)
NP = NG * BM
PP = NP // 2

RP = 256
NW = 32
DEPTH = 3
CHUNKS = [10, 10, 10, 10, 9]


def _sc_gather_body(nbp, maxj, ids_e_hbm, ids_o_hbm, table_hbm, emb2_hbm,
                    idx_v, lo_v, hi_v, *sems):
    NBP, MAXJ = nbp, maxj
    wid = lax.axis_index("s") * 2 + lax.axis_index("c")

    def sem(p, k):
        return sems[p * 5 + k]

    def blk(j):
        return wid + j * NW

    def front(j):
        p = j % DEPTH

        @pl.when(blk(j) < NBP)
        def _():
            base = blk(j) * RP
            pltpu.make_async_copy(
                ids_e_hbm.at[pl.ds(base, RP)], idx_v.at[p, pl.ds(0, RP)],
                sem(p, 0)).start()
            pltpu.make_async_copy(
                ids_o_hbm.at[pl.ds(base, RP)], idx_v.at[p, pl.ds(RP, RP)],
                sem(p, 0)).start()

    def mid(j):
        p = j % DEPTH

        @pl.when(blk(j) < NBP)
        def _():
            base = blk(j) * RP
            pltpu.make_async_copy(
                ids_e_hbm.at[pl.ds(base, RP)], idx_v.at[p, pl.ds(0, RP)],
                sem(p, 0)).wait()
            pltpu.make_async_copy(
                ids_o_hbm.at[pl.ds(base, RP)], idx_v.at[p, pl.ds(RP, RP)],
                sem(p, 0)).wait()
            pltpu.make_async_copy(
                table_hbm.at[idx_v.at[p, pl.ds(0, RP)]],
                lo_v.at[p], sem(p, 1)).start()
            pltpu.make_async_copy(
                table_hbm.at[idx_v.at[p, pl.ds(RP, RP)]],
                hi_v.at[p], sem(p, 2)).start()

    def back(j):
        p = j % DEPTH

        @pl.when(blk(j) < NBP)
        def _():
            base = blk(j) * RP
            pltpu.make_async_copy(
                table_hbm.at[idx_v.at[p, pl.ds(0, RP)]],
                lo_v.at[p], sem(p, 1)).wait()
            pltpu.make_async_copy(
                table_hbm.at[idx_v.at[p, pl.ds(RP, RP)]],
                hi_v.at[p], sem(p, 2)).wait()
            pltpu.make_async_copy(
                lo_v.at[p],
                emb2_hbm.at[pl.ds(base, RP), pl.ds(0, EMB_DIM)],
                sem(p, 3)).start()
            pltpu.make_async_copy(
                hi_v.at[p],
                emb2_hbm.at[pl.ds(base, RP), pl.ds(EMB_DIM, EMB_DIM)],
                sem(p, 4)).start()

    def drain(j):
        p = j % DEPTH

        @pl.when(blk(j) < NBP)
        def _():
            base = blk(j) * RP
            pltpu.make_async_copy(
                lo_v.at[p],
                emb2_hbm.at[pl.ds(base, RP), pl.ds(0, EMB_DIM)],
                sem(p, 3)).wait()
            pltpu.make_async_copy(
                hi_v.at[p],
                emb2_hbm.at[pl.ds(base, RP), pl.ds(EMB_DIM, EMB_DIM)],
                sem(p, 4)).wait()

    for step in range(MAXJ + DEPTH):
        jd = step - DEPTH
        if 0 <= jd < MAXJ:
            drain(jd)
        if step < MAXJ:
            front(step)
        jm = step - 1
        if 0 <= jm < MAXJ:
            mid(jm)
        jb = step - 2
        if 0 <= jb < MAXJ:
            back(jb)


def _sc_gather(ids_e, ids_o, embed_weight):
    import functools
    pp = ids_e.shape[0]
    nbp = pp // RP
    maxj = -(-nbp // NW)
    mesh = plsc.VectorSubcoreMesh(core_axis_name="c", subcore_axis_name="s")
    return pl.kernel(
        functools.partial(_sc_gather_body, nbp, maxj),
        mesh=mesh,
        out_type=jax.ShapeDtypeStruct((pp, 2 * EMB_DIM), jnp.float32),
        scratch_types=[
            pltpu.VMEM((DEPTH, 2 * RP), jnp.int32),
            pltpu.VMEM((DEPTH, RP, EMB_DIM), jnp.float32),
            pltpu.VMEM((DEPTH, RP, EMB_DIM), jnp.float32),
        ] + [pltpu.SemaphoreType.DMA] * (DEPTH * 5),
        compiler_params=pltpu.CompilerParams(use_tc_tiling_on_sc=False),
    )(ids_e, ids_o, embed_weight)


def _tc_concat_body(emb2_ref, feats_ref, *rest):
    out_ref = rest[-1]
    e = emb2_ref[...]
    f = feats_ref[...]
    et = e.T
    emb_part = jnp.concatenate(
        [et[:EMB_DIM, :], et[EMB_DIM:, :]], axis=1)
    out_ref[...] = jnp.concatenate(
        [emb_part, f.T], axis=0)


def _tc_concat(b0, nb, emb2_c, feats, out_prev):
    in_specs = [
        pl.BlockSpec((HB, 2 * EMB_DIM), lambda i: (i, 0)),
        pl.BlockSpec((BM, D_FEAT), lambda i: (b0 + i, 0)),
    ]
    args = [emb2_c, feats]
    alias = {}
    if out_prev is not None:
        in_specs.append(pl.BlockSpec(memory_space=pl.ANY))
        args.append(out_prev)
        alias = {2: 0}
    return pl.pallas_call(
        _tc_concat_body,
        grid=(nb,),
        in_specs=in_specs,
        out_specs=pl.BlockSpec((OUT_DIM, BM), lambda i: (0, b0 + i)),
        out_shape=jax.ShapeDtypeStruct((OUT_DIM, N), jnp.float32),
        input_output_aliases=alias,
    )(*args)


@jax.jit
def _feature_prep(ids, feats, embed_weight):
    idsb = jnp.pad(ids, (0, NP - N)).reshape(NG, BM)
    lo = idsb[:, :HB].reshape(-1)
    hi = idsb[:, HB:].reshape(-1)
    emb2 = []
    b0 = 0
    for nb in CHUNKS:
        emb2.append(_sc_gather(lo[b0 * HB:(b0 + nb) * HB],
                               hi[b0 * HB:(b0 + nb) * HB], embed_weight))
        b0 += nb
    out_t = None
    b0 = 0
    for c, nb in enumerate(CHUNKS):
        out_t = _tc_concat(b0, nb, emb2[c], feats, out_t)
        b0 += nb
    return out_t.T


def kernel(ids, feats, embed_weight):
    return _feature_prep(ids.astype(jnp.int32), feats, embed_weight)


# speedup vs baseline: 4.3393x; 1.0163x over previous
import jax
import jax.numpy as jnp
from jax import lax
from jax.experimental import pallas as pl
from jax.experimental.pallas import tpu as pltpu
from jax.experimental.pallas import tpu_sc as plsc

N = 100000
EMB_DIM = 64
D_FEAT = 128
OUT_DIM = EMB_DIM + D_FEAT
BM = 2048
HB = BM // 2
NG = -(-N // BM)
NP = NG * BM
PP = NP // 2

RP = 256
NW = 32
DEPTH = 3
CHUNKS = [5, 11, 11, 11, 11]


def _sc_gather_body(b0, nbp, maxj, ids_hbm, table_hbm, emb2_hbm,
                    idx_v, lo_v, hi_v, *sems):
    NBP, MAXJ = nbp, maxj
    SB = BM // RP
    wid = lax.axis_index("s") * 2 + lax.axis_index("c")

    def sem(p, k):
        return sems[p * 5 + k]

    def blk(j):
        return wid + j * NW

    def lo_base(j):
        g = b0 * (HB // RP) + blk(j)
        return (g // (HB // RP)) * BM + (g % (HB // RP)) * RP

    def front(j):
        p = j % DEPTH

        @pl.when(blk(j) < NBP)
        def _():
            lob = lo_base(j)
            pltpu.make_async_copy(
                ids_hbm.at[pl.ds(lob, RP)], idx_v.at[p, pl.ds(0, RP)],
                sem(p, 0)).start()
            pltpu.make_async_copy(
                ids_hbm.at[pl.ds(lob + HB, RP)], idx_v.at[p, pl.ds(RP, RP)],
                sem(p, 0)).start()

    def mid(j):
        p = j % DEPTH

        @pl.when(blk(j) < NBP)
        def _():
            lob = lo_base(j)
            pltpu.make_async_copy(
                ids_hbm.at[pl.ds(lob, RP)], idx_v.at[p, pl.ds(0, RP)],
                sem(p, 0)).wait()
            pltpu.make_async_copy(
                ids_hbm.at[pl.ds(lob + HB, RP)], idx_v.at[p, pl.ds(RP, RP)],
                sem(p, 0)).wait()
            pltpu.make_async_copy(
                table_hbm.at[idx_v.at[p, pl.ds(0, RP)]],
                lo_v.at[p], sem(p, 1)).start()
            pltpu.make_async_copy(
                table_hbm.at[idx_v.at[p, pl.ds(RP, RP)]],
                hi_v.at[p], sem(p, 2)).start()

    def back(j):
        p = j % DEPTH

        @pl.when(blk(j) < NBP)
        def _():
            base = blk(j) * RP
            pltpu.make_async_copy(
                table_hbm.at[idx_v.at[p, pl.ds(0, RP)]],
                lo_v.at[p], sem(p, 1)).wait()
            pltpu.make_async_copy(
                table_hbm.at[idx_v.at[p, pl.ds(RP, RP)]],
                hi_v.at[p], sem(p, 2)).wait()
            pltpu.make_async_copy(
                lo_v.at[p],
                emb2_hbm.at[pl.ds(base, RP), pl.ds(0, EMB_DIM)],
                sem(p, 3)).start()
            pltpu.make_async_copy(
                hi_v.at[p],
                emb2_hbm.at[pl.ds(base, RP), pl.ds(EMB_DIM, EMB_DIM)],
                sem(p, 4)).start()

    def drain(j):
        p = j % DEPTH

        @pl.when(blk(j) < NBP)
        def _():
            base = blk(j) * RP
            pltpu.make_async_copy(
                lo_v.at[p],
                emb2_hbm.at[pl.ds(base, RP), pl.ds(0, EMB_DIM)],
                sem(p, 3)).wait()
            pltpu.make_async_copy(
                hi_v.at[p],
                emb2_hbm.at[pl.ds(base, RP), pl.ds(EMB_DIM, EMB_DIM)],
                sem(p, 4)).wait()

    for step in range(MAXJ + DEPTH):
        jd = step - DEPTH
        if 0 <= jd < MAXJ:
            drain(jd)
        if step < MAXJ:
            front(step)
        jm = step - 1
        if 0 <= jm < MAXJ:
            mid(jm)
        jb = step - 2
        if 0 <= jb < MAXJ:
            back(jb)


def _sc_gather(b0, nb, ids_p, embed_weight):
    import functools
    pp = nb * HB
    nbp = pp // RP
    maxj = -(-nbp // NW)
    mesh = plsc.VectorSubcoreMesh(core_axis_name="c", subcore_axis_name="s")
    return pl.kernel(
        functools.partial(_sc_gather_body, b0, nbp, maxj),
        mesh=mesh,
        out_type=jax.ShapeDtypeStruct((pp, 2 * EMB_DIM), jnp.float32),
        scratch_types=[
            pltpu.VMEM((DEPTH, 2 * RP), jnp.int32),
            pltpu.VMEM((DEPTH, RP, EMB_DIM), jnp.float32),
            pltpu.VMEM((DEPTH, RP, EMB_DIM), jnp.float32),
        ] + [pltpu.SemaphoreType.DMA] * (DEPTH * 5),
        compiler_params=pltpu.CompilerParams(use_tc_tiling_on_sc=False),
    )(ids_p, embed_weight)


def _tc_concat_body(emb2_ref, feats_ref, *rest):
    out_ref = rest[-1]
    e = emb2_ref[...]
    f = feats_ref[...]
    et = e.T
    emb_part = jnp.concatenate(
        [et[:EMB_DIM, :], et[EMB_DIM:, :]], axis=1)
    out_ref[...] = jnp.concatenate(
        [emb_part, f.T], axis=0)


def _tc_concat(b0, nb, emb2_c, feats, out_prev):
    in_specs = [
        pl.BlockSpec((HB, 2 * EMB_DIM), lambda i: (i, 0)),
        pl.BlockSpec((BM, D_FEAT), lambda i: (b0 + i, 0)),
    ]
    args = [emb2_c, feats]
    alias = {}
    if out_prev is not None:
        in_specs.append(pl.BlockSpec(memory_space=pl.ANY))
        args.append(out_prev)
        alias = {2: 0}
    return pl.pallas_call(
        _tc_concat_body,
        grid=(nb,),
        in_specs=in_specs,
        out_specs=pl.BlockSpec((OUT_DIM, BM), lambda i: (0, b0 + i)),
        out_shape=jax.ShapeDtypeStruct((OUT_DIM, N), jnp.float32),
        input_output_aliases=alias,
    )(*args)


@jax.jit
def _feature_prep(ids, feats, embed_weight):
    ids_p = jnp.pad(ids, (0, NP - N))
    emb2 = []
    b0 = 0
    for nb in CHUNKS:
        emb2.append(_sc_gather(b0, nb, ids_p, embed_weight))
        b0 += nb
    out_t = None
    b0 = 0
    for c, nb in enumerate(CHUNKS):
        out_t = _tc_concat(b0, nb, emb2[c], feats, out_t)
        b0 += nb
    return out_t.T


def kernel(ids, feats, embed_weight):
    return _feature_prep(ids.astype(jnp.int32), feats, embed_weight)
